# baseline (speedup 1.0000x reference)
import functools

import jax
import jax.numpy as jnp
from jax import lax
from jax.experimental import pallas as pl
from jax.experimental.pallas import tpu as pltpu
from jax.experimental.pallas import tpu_sc as plsc

_CHUNK = 128


@functools.cache
def _build(B, V, D, NC, NS):
    NW = NC * NS
    b_per_w = B // NW
    n_chunk = b_per_w // _CHUNK

    mesh = plsc.VectorSubcoreMesh(core_axis_name="c", subcore_axis_name="s")

    @functools.partial(
        pl.kernel,
        mesh=mesh,
        out_type=jax.ShapeDtypeStruct((B, D), jnp.float32),
        scratch_types=[
            pltpu.VMEM((n_chunk, _CHUNK), jnp.int32),
            pltpu.VMEM((b_per_w, D), jnp.float32),
            pltpu.SemaphoreType.DMA,
        ],
        compiler_params=pltpu.CompilerParams(use_tc_tiling_on_sc=False),
    )
    def gather_kernel(idx_hbm, table_hbm, out_hbm, idx_v, rows_v, sem):
        wid = lax.axis_index("s") * NC + lax.axis_index("c")
        pltpu.sync_copy(idx_hbm.at[wid], idx_v)
        copies = []
        for j in range(n_chunk):
            copies.append(
                pltpu.async_copy(
                    table_hbm.at[idx_v.at[j]],
                    rows_v.at[pl.ds(j * _CHUNK, _CHUNK)],
                    sem,
                )
            )
        for c in copies:
            c.wait()
        pltpu.sync_copy(rows_v, out_hbm.at[pl.ds(wid * b_per_w, b_per_w)])

    return gather_kernel


def kernel(users, user_embed):
    B, = users.shape
    V, D = user_embed.shape
    info = plsc.get_sparse_core_info()
    NC, NS = info.num_cores, info.num_subcores
    NW = NC * NS
    b_per_w = B // NW
    idx = users.astype(jnp.int32).reshape(NW, b_per_w // _CHUNK, _CHUNK)
    return _build(B, V, D, NC, NS)(idx, user_embed)


# speedup vs baseline: 1.6418x; 1.6418x over previous
import functools

import jax
import jax.numpy as jnp
from jax import lax
from jax.experimental import pallas as pl
from jax.experimental.pallas import tpu as pltpu
from jax.experimental.pallas import tpu_sc as plsc

_W = 24


@functools.cache
def _build(B, V, D, NC, NS):
    NW = NC * NS
    n = B // NW

    mesh = plsc.VectorSubcoreMesh(core_axis_name="c", subcore_axis_name="s")

    @functools.partial(
        pl.kernel,
        mesh=mesh,
        out_type=jax.ShapeDtypeStruct((B, D), jnp.float32),
        scratch_types=[
            pltpu.VMEM((n,), jnp.int32),
            pltpu.VMEM((n, D), jnp.float32),
            pltpu.SemaphoreType.DMA,
        ],
    )
    def gather_kernel(idx_hbm, table_hbm, out_hbm, idx_v, rows_v, sem):
        wid = lax.axis_index("s") * NC + lax.axis_index("c")
        base = wid * n
        pltpu.sync_copy(idx_hbm.at[pl.ds(base, n)], idx_v)

        L = 16
        n_grp = n // L

        def issue_group(g):
            v = idx_v[pl.ds(g * L, L)]
            for j in range(L):
                pltpu.async_copy(
                    table_hbm.at[v[j]], rows_v.at[g * L + j], sem
                )

        def drain_group():
            for _ in range(L):
                pltpu.make_async_copy(table_hbm.at[0], rows_v.at[0], sem).wait()

        issue_group(0)

        def body(g, carry):
            drain_group()
            issue_group(g)
            return carry

        lax.fori_loop(1, n_grp, body, 0, unroll=False)
        drain_group()

        pltpu.sync_copy(rows_v, out_hbm.at[pl.ds(base, n)])

    return gather_kernel


def kernel(users, user_embed):
    B, = users.shape
    V, D = user_embed.shape
    info = plsc.get_sparse_core_info()
    return _build(B, V, D, info.num_cores, info.num_subcores)(
        users.astype(jnp.int32), user_embed
    )


# speedup vs baseline: 1.7250x; 1.0506x over previous
import functools

import jax
import jax.numpy as jnp
from jax import lax
from jax.experimental import pallas as pl
from jax.experimental.pallas import tpu as pltpu
from jax.experimental.pallas import tpu_sc as plsc

_W = 24


@functools.cache
def _build(B, V, D, NC, NS):
    NW = NC * NS
    n = B // NW

    mesh = plsc.VectorSubcoreMesh(core_axis_name="c", subcore_axis_name="s")

    @functools.partial(
        pl.kernel,
        mesh=mesh,
        out_type=jax.ShapeDtypeStruct((B, D), jnp.float32),
        scratch_types=[
            pltpu.VMEM((n,), jnp.int32),
            pltpu.VMEM((n, D), jnp.float32),
            pltpu.SemaphoreType.DMA,
        ],
    )
    def gather_kernel(idx_hbm, table_hbm, out_hbm, idx_v, rows_v, sem):
        wid = lax.axis_index("s") * NC + lax.axis_index("c")
        base = wid * n
        pltpu.sync_copy(idx_hbm.at[pl.ds(base, n)], idx_v)

        L = 16
        n_grp = n // L

        def body(g, carry):
            v = idx_v[pl.ds(g * L, L)]
            for j in range(L):
                pltpu.async_copy(
                    table_hbm.at[v[j]], rows_v.at[g * L + j], sem
                )
            return carry

        lax.fori_loop(0, n_grp, body, 0, unroll=False)
        pltpu.make_async_copy(table_hbm.at[pl.ds(0, n)], rows_v, sem).wait()

        pltpu.sync_copy(rows_v, out_hbm.at[pl.ds(base, n)])

    return gather_kernel


def kernel(users, user_embed):
    B, = users.shape
    V, D = user_embed.shape
    info = plsc.get_sparse_core_info()
    return _build(B, V, D, info.num_cores, info.num_subcores)(
        users.astype(jnp.int32), user_embed
    )
